# Initial kernel scaffold; baseline (speedup 1.0000x reference)
import jax
import jax.numpy as jnp
from jax.experimental import pallas as pl


def kernel(x, neighbors, W_self, b_self, W_neigh, b_neigh, W_coarsen, b_coarsen):
    raise NotImplementedError("write your pallas kernel here")


# speedup vs baseline: 1.5740x; 1.5740x over previous
import functools

import jax
import jax.numpy as jnp
from jax import lax
from jax.experimental import pallas as pl
from jax.experimental.pallas import tpu as pltpu
from jax.experimental.pallas import tpu_sc as plsc

_N = 10000
_D = 16
_S = 8
_DIN = 256
_DOUT = 512
_NC = 2
_NS = 16
_NW = _NC * _NS
_B = 8
_CH = 40
_PERW = _B * _CH
_NPAD = _NW * _PERW
_COLS = _DIN // 16


def _agg_body(nbr_hbm, x_hbm, s_out, c_out, idx_v, rows_v, sv, cv, sem0, sem1):
    wid = lax.axis_index("s") * _NC + lax.axis_index("c")
    base = wid * _PERW

    def issue(g, b):
        sem = sem0 if b == 0 else sem1
        node0 = pl.multiple_of((base + g * _B) * _D, 128)
        pltpu.sync_copy(nbr_hbm.at[pl.ds(node0, _B * _D)], idx_v.at[b])
        pltpu.make_async_copy(x_hbm.at[idx_v.at[b]], rows_v.at[b], sem).start()

    def wait(b):
        sem = sem0 if b == 0 else sem1
        pltpu.make_async_copy(x_hbm.at[idx_v.at[b]], rows_v.at[b], sem).wait()

    issue(0, 0)

    def outer(i, carry):
        for b in range(2):
            g = i * 2 + b
            nxt = g + 1

            @pl.when(nxt < _CH)
            def _():
                issue(nxt, 1 - b)

            wait(b)
            rv = rows_v.at[b]
            node0 = base + g * _B

            def node(n, c2):
                r0 = n * _D
                for col in range(_COLS):
                    sl = pl.ds(col * 16, 16)
                    s01 = rv[r0 + 0, sl] + rv[r0 + 1, sl]
                    s23 = rv[r0 + 2, sl] + rv[r0 + 3, sl]
                    s45 = rv[r0 + 4, sl] + rv[r0 + 5, sl]
                    s67 = rv[r0 + 6, sl] + rv[r0 + 7, sl]
                    sv[n, sl] = ((s01 + s23) + (s45 + s67)) * 0.125
                    c01 = rv[r0 + 8, sl] + rv[r0 + 9, sl]
                    c23 = rv[r0 + 10, sl] + rv[r0 + 11, sl]
                    c45 = rv[r0 + 12, sl] + rv[r0 + 13, sl]
                    c67 = rv[r0 + 14, sl] + rv[r0 + 15, sl]
                    cv[n, sl] = ((c01 + c23) + (c45 + c67)) * 0.125
                return c2

            lax.fori_loop(0, _B, node, 0)
            pltpu.sync_copy(sv, s_out.at[pl.ds(node0, _B)])
            pltpu.sync_copy(cv, c_out.at[pl.ds(node0, _B)])
        return carry

    lax.fori_loop(0, _CH // 2, outer, 0)


_agg = pl.kernel(
    _agg_body,
    out_type=[
        jax.ShapeDtypeStruct((_NPAD, _DIN), jnp.float32),
        jax.ShapeDtypeStruct((_NPAD, _DIN), jnp.float32),
    ],
    mesh=plsc.VectorSubcoreMesh(
        core_axis_name="c", subcore_axis_name="s", num_cores=_NC,
        num_subcores=_NS),
    scratch_types=[
        pltpu.VMEM((2, _B * _D), jnp.int32),
        pltpu.VMEM((2, _B * _D, _DIN), jnp.float32),
        pltpu.VMEM((_B, _DIN), jnp.float32),
        pltpu.VMEM((_B, _DIN), jnp.float32),
        pltpu.SemaphoreType.DMA,
        pltpu.SemaphoreType.DMA,
    ],
)


def _gemm_body(x_ref, s_ref, c_ref, ws, wn, wc, bs, bn, bc, o_ref):
    acc = jnp.dot(x_ref[...], ws[...], preferred_element_type=jnp.float32)
    acc = acc + jnp.dot(s_ref[...], wn[...], preferred_element_type=jnp.float32)
    acc = acc + jnp.dot(c_ref[...], wc[...], preferred_element_type=jnp.float32)
    o_ref[...] = acc + (bs[...] + bn[...] + bc[...])


_BM = 1000


def _fused_gemm(x, s_agg, c_agg, ws, wn, wc, bs, bn, bc):
    grid = (_N // _BM,)
    row_spec = pl.BlockSpec((_BM, _DIN), lambda i: (i, 0))
    w_spec = pl.BlockSpec((_DIN, _DOUT), lambda i: (0, 0))
    b_spec = pl.BlockSpec((1, _DOUT), lambda i: (0, 0))
    return pl.pallas_call(
        _gemm_body,
        grid=grid,
        in_specs=[row_spec, row_spec, row_spec, w_spec, w_spec, w_spec,
                  b_spec, b_spec, b_spec],
        out_specs=pl.BlockSpec((_BM, _DOUT), lambda i: (i, 0)),
        out_shape=jax.ShapeDtypeStruct((_N, _DOUT), jnp.float32),
    )(x, s_agg, c_agg, ws, wn, wc, bs, bn, bc)


def kernel(x, neighbors, W_self, b_self, W_neigh, b_neigh, W_coarsen,
           b_coarsen):
    nbr_pad = jnp.pad(neighbors, ((0, _NPAD - _N), (0, 0)))
    s_agg, c_agg = _agg(nbr_pad.reshape(-1), x)
    return _fused_gemm(
        x, s_agg, c_agg, W_self, W_neigh, W_coarsen,
        b_self.reshape(1, -1), b_neigh.reshape(1, -1),
        b_coarsen.reshape(1, -1))
